# Initial kernel scaffold; baseline (speedup 1.0000x reference)
import jax
import jax.numpy as jnp
from jax.experimental import pallas as pl


def kernel(item_embeddings, neighbors, protein_embedding, agg_W, agg_b):
    raise NotImplementedError("write your pallas kernel here")


# speedup vs baseline: 4.4806x; 4.4806x over previous
import functools

import jax
import jax.numpy as jnp
from jax import lax
from jax.experimental import pallas as pl
from jax.experimental.pallas import tpu as pltpu
from jax.experimental.pallas import tpu_sc as plsc

N_HOP = 2
N_MEMORY = 32
EMB_DIM = 64
BATCH = 16384
ROWS = N_HOP * BATCH * N_MEMORY
NW = 32
PER_W = ROWS // NW
CHUNK = 512
SUB = 128
N_CHUNK = PER_W // CHUNK

BB = 512


PAIRS_W = PER_W // 2
PCHUNK = 256
NSUB = PCHUNK // SUB
NCH = PAIRS_W // PCHUNK


def _sc_gather_body(table_hbm, idx_ev_hbm, idx_od_hbm, out_hbm, idx_ev_v, idx_od_v, rows_v, gsem, asem, wsem):
    wid = lax.axis_index("s") * 2 + lax.axis_index("c")
    base = wid * PAIRS_W
    pltpu.sync_copy(idx_ev_hbm.at[pl.ds(base, PAIRS_W)], idx_ev_v)
    pltpu.sync_copy(idx_od_hbm.at[pl.ds(base, PAIRS_W)], idx_od_v)

    def e_descs(c, buf):
        return [
            pltpu.make_async_copy(
                table_hbm.at[idx_ev_v.at[pl.ds(c * PCHUNK + s * SUB, SUB)]],
                rows_v.at[buf, pl.ds(s * SUB, SUB)],
                gsem,
            )
            for s in range(NSUB)
        ]

    def o_descs(c, buf):
        return [
            pltpu.make_async_copy(
                table_hbm.at[idx_od_v.at[pl.ds(c * PCHUNK + s * SUB, SUB)]],
                rows_v.at[buf, pl.ds(s * SUB, SUB)],
                asem,
            )
            for s in range(NSUB)
        ]

    def w_desc(c, buf):
        return pltpu.make_async_copy(
            rows_v.at[buf],
            out_hbm.at[pl.ds(base + c * PCHUNK, PCHUNK)],
            wsem,
        )

    def start_e(c, buf):
        for d in e_descs(c, buf):
            d.start()

    def start_o(c, buf):
        for d in o_descs(c, buf):
            d.start(add=True)

    def wait_e(c, buf):
        for d in e_descs(c, buf):
            d.wait()

    def wait_o(c, buf):
        for d in o_descs(c, buf):
            d.wait()

    start_e(0, 0)

    def body(i, carry):
        a = 2 * i
        b = a + 1
        wait_e(a, 0)
        start_o(a, 0)

        @pl.when(i > 0)
        def _():
            w_desc(b - 2, 1).wait()

        start_e(b, 1)
        wait_o(a, 0)
        w_desc(a, 0).start()
        wait_e(b, 1)
        start_o(b, 1)
        w_desc(a, 0).wait()

        @pl.when(i < NCH // 2 - 1)
        def _():
            start_e(a + 2, 0)

        wait_o(b, 1)
        w_desc(b, 1).start()
        return carry

    lax.fori_loop(0, NCH // 2, body, 0)
    w_desc(NCH - 1, 1).wait()


def _sc_gather(table_eo, idx_even, idx_odd):
    mesh = plsc.VectorSubcoreMesh(core_axis_name="c", subcore_axis_name="s")
    kern = functools.partial(
        pl.kernel,
        out_type=jax.ShapeDtypeStruct((ROWS // 2, 2 * EMB_DIM), jnp.float32),
        mesh=mesh,
        scratch_types=[
            pltpu.VMEM((PAIRS_W,), jnp.int32),
            pltpu.VMEM((PAIRS_W,), jnp.int32),
            pltpu.VMEM((2, PCHUNK, 2 * EMB_DIM), jnp.float32),
            pltpu.SemaphoreType.DMA,
            pltpu.SemaphoreType.DMA,
            pltpu.SemaphoreType.DMA,
        ],
    )(_sc_gather_body)
    return kern(table_eo, idx_even, idx_odd)


def _attn_body(item_ref, g0_ref, g1_ref, w_ref, b_ref, sel_ref, out_ref):
    x = item_ref[...]
    x2 = jnp.concatenate([x, x], axis=1)
    w = w_ref[...]
    sel = sel_ref[...]

    M2 = N_MEMORY // 2
    D2 = 2 * EMB_DIM

    def hop(g, xin):
        p = (g * xin[:, None, :]).reshape(BB * M2, D2)
        c_wide = lax.dot_general(
            p, sel, (((1,), (0,)), ((), ())),
            precision=lax.Precision.DEFAULT, preferred_element_type=jnp.float32,
        ).reshape(BB, M2, D2)
        e = jnp.exp(c_wide)
        es = jnp.sum(e, axis=1)
        st = es[:, :EMB_DIM] + es[:, EMB_DIM:]
        rinv = 1.0 / st
        r2 = jnp.concatenate([rinv, rinv], axis=1)
        t = jnp.sum(g * e, axis=1) * r2
        return t[:, :EMB_DIM] + t[:, EMB_DIM:]

    i0 = hop(g0_ref[...], x2)
    i1 = hop(g1_ref[...], jnp.concatenate([i0, i0], axis=1))
    out_ref[...] = (
        lax.dot_general(
            i0, w[:EMB_DIM], (((1,), (0,)), ((), ())),
            precision=lax.Precision.HIGHEST, preferred_element_type=jnp.float32,
        )
        + lax.dot_general(
            i1, w[EMB_DIM:], (((1,), (0,)), ((), ())),
            precision=lax.Precision.HIGHEST, preferred_element_type=jnp.float32,
        )
        + b_ref[...]
    )


def _sel_matrix():
    col = jnp.arange(2 * EMB_DIM)[None, :]
    row = jnp.arange(2 * EMB_DIM)[:, None]
    return jnp.where((col < EMB_DIM) == (row < EMB_DIM), 1.0, 0.0).astype(
        jnp.float32
    )


def _attention_aggregate(item_embeddings, g0, g1, agg_W, agg_b):
    batch = item_embeddings.shape[0]
    grid = (batch // BB,)
    return pl.pallas_call(
        _attn_body,
        grid=grid,
        in_specs=[
            pl.BlockSpec((BB, EMB_DIM), lambda i: (i, 0)),
            pl.BlockSpec((BB, N_MEMORY // 2, 2 * EMB_DIM), lambda i: (i, 0, 0)),
            pl.BlockSpec((BB, N_MEMORY // 2, 2 * EMB_DIM), lambda i: (i, 0, 0)),
            pl.BlockSpec((N_HOP * EMB_DIM, EMB_DIM), lambda i: (0, 0)),
            pl.BlockSpec((1, EMB_DIM), lambda i: (0, 0)),
            pl.BlockSpec((2 * EMB_DIM, 2 * EMB_DIM), lambda i: (0, 0)),
        ],
        out_specs=pl.BlockSpec((BB, EMB_DIM), lambda i: (i, 0)),
        out_shape=jax.ShapeDtypeStruct((batch, EMB_DIM), jnp.float32),
    )(item_embeddings, g0, g1, agg_W, agg_b, _sel_matrix())


def kernel(item_embeddings, neighbors, protein_embedding, agg_W, agg_b):
    idx_flat = neighbors.astype(jnp.int32).reshape(-1)
    n_prot = protein_embedding.shape[0]
    zeros = jnp.zeros_like(protein_embedding)
    table_eo = jnp.concatenate(
        [
            jnp.concatenate([protein_embedding, zeros], axis=1),
            jnp.concatenate([zeros, protein_embedding], axis=1),
        ],
        axis=0,
    )
    gathered = _sc_gather(
        table_eo, idx_flat[0::2], idx_flat[1::2] + n_prot
    )
    half = ROWS // (2 * N_HOP)
    g0 = gathered[:half].reshape(BATCH, N_MEMORY // 2, 2 * EMB_DIM)
    g1 = gathered[half:].reshape(BATCH, N_MEMORY // 2, 2 * EMB_DIM)
    return _attention_aggregate(
        item_embeddings, g0, g1, agg_W, agg_b.reshape(1, EMB_DIM)
    )


# speedup vs baseline: 5.4389x; 1.2139x over previous
import functools

import jax
import jax.numpy as jnp
from jax import lax
from jax.experimental import pallas as pl
from jax.experimental.pallas import tpu as pltpu
from jax.experimental.pallas import tpu_sc as plsc

N_HOP = 2
N_MEMORY = 32
EMB_DIM = 64
BATCH = 16384
ROWS = N_HOP * BATCH * N_MEMORY
NW = 32
PER_W = ROWS // NW
CHUNK = 512
SUB = 128
N_CHUNK = PER_W // CHUNK

BB = 512


PAIRS_W = PER_W // 2
PCHUNK = 256
NSUB = PCHUNK // SUB
NCH = PAIRS_W // PCHUNK


def _sc_gather_body(table_hbm, idx_ev_hbm, idx_od_hbm, out_hbm, idx_ev_v, idx_od_v, rows_v, gsem, asem, wsem):
    wid = lax.axis_index("s") * 2 + lax.axis_index("c")
    base = wid * PAIRS_W
    pltpu.sync_copy(idx_ev_hbm.at[pl.ds(base, PAIRS_W)], idx_ev_v)
    pltpu.sync_copy(idx_od_hbm.at[pl.ds(base, PAIRS_W)], idx_od_v)

    def e_descs(c, buf):
        return [
            pltpu.make_async_copy(
                table_hbm.at[idx_ev_v.at[pl.ds(c * PCHUNK + s * SUB, SUB)]],
                rows_v.at[buf, pl.ds(s * SUB, SUB)],
                gsem,
            )
            for s in range(NSUB)
        ]

    def o_descs(c, buf):
        return [
            pltpu.make_async_copy(
                table_hbm.at[idx_od_v.at[pl.ds(c * PCHUNK + s * SUB, SUB)]],
                rows_v.at[buf, pl.ds(s * SUB, SUB)],
                asem,
            )
            for s in range(NSUB)
        ]

    def w_desc(c, buf):
        return pltpu.make_async_copy(
            rows_v.at[buf],
            out_hbm.at[pl.ds(base + c * PCHUNK, PCHUNK)],
            wsem,
        )

    def start_e(c, buf):
        for d in e_descs(c, buf):
            d.start()

    def start_o(c, buf):
        for d in o_descs(c, buf):
            d.start(add=True)

    def wait_e(c, buf):
        for d in e_descs(c, buf):
            d.wait()

    def wait_o(c, buf):
        for d in o_descs(c, buf):
            d.wait()

    start_e(0, 0)

    def body(i, carry):
        a = 2 * i
        b = a + 1
        wait_e(a, 0)
        start_o(a, 0)

        @pl.when(i > 0)
        def _():
            w_desc(b - 2, 1).wait()

        start_e(b, 1)
        wait_o(a, 0)
        w_desc(a, 0).start()
        wait_e(b, 1)
        start_o(b, 1)
        w_desc(a, 0).wait()

        @pl.when(i < NCH // 2 - 1)
        def _():
            start_e(a + 2, 0)

        wait_o(b, 1)
        w_desc(b, 1).start()
        return carry

    lax.fori_loop(0, NCH // 2, body, 0)
    w_desc(NCH - 1, 1).wait()


def _sc_gather(table_eo, idx_even, idx_odd):
    mesh = plsc.VectorSubcoreMesh(core_axis_name="c", subcore_axis_name="s")
    kern = functools.partial(
        pl.kernel,
        out_type=jax.ShapeDtypeStruct((ROWS // 2, 2 * EMB_DIM), jnp.float32),
        mesh=mesh,
        scratch_types=[
            pltpu.VMEM((PAIRS_W,), jnp.int32),
            pltpu.VMEM((PAIRS_W,), jnp.int32),
            pltpu.VMEM((2, PCHUNK, 2 * EMB_DIM), jnp.float32),
            pltpu.SemaphoreType.DMA,
            pltpu.SemaphoreType.DMA,
            pltpu.SemaphoreType.DMA,
        ],
    )(_sc_gather_body)
    return kern(table_eo, idx_even, idx_odd)


def _attn_body(item_ref, g0_ref, g1_ref, w_ref, b_ref, sel_ref, out_ref):
    x = item_ref[...]
    x2 = jnp.concatenate([x, x], axis=1)
    w = w_ref[...]
    sel = sel_ref[...]

    M2 = N_MEMORY // 2
    D2 = 2 * EMB_DIM

    def hop(g, xin):
        p = (g * xin[:, None, :]).reshape(BB * M2, D2)
        c_wide = lax.dot_general(
            p, sel, (((1,), (0,)), ((), ())),
            precision=lax.Precision.DEFAULT, preferred_element_type=jnp.float32,
        ).reshape(BB, M2, D2)
        e = jnp.exp(c_wide)
        es = jnp.sum(e, axis=1)
        st = es[:, :EMB_DIM] + es[:, EMB_DIM:]
        rinv = 1.0 / st
        r2 = jnp.concatenate([rinv, rinv], axis=1)
        t = jnp.sum(g * e, axis=1) * r2
        return t[:, :EMB_DIM] + t[:, EMB_DIM:]

    i0 = hop(g0_ref[0], x2)
    i1 = hop(g1_ref[0], jnp.concatenate([i0, i0], axis=1))
    out_ref[...] = (
        lax.dot_general(
            i0, w[:EMB_DIM], (((1,), (0,)), ((), ())),
            precision=lax.Precision.HIGHEST, preferred_element_type=jnp.float32,
        )
        + lax.dot_general(
            i1, w[EMB_DIM:], (((1,), (0,)), ((), ())),
            precision=lax.Precision.HIGHEST, preferred_element_type=jnp.float32,
        )
        + b_ref[...]
    )


def _sel_matrix():
    col = jnp.arange(2 * EMB_DIM)[None, :]
    row = jnp.arange(2 * EMB_DIM)[:, None]
    return jnp.where((col < EMB_DIM) == (row < EMB_DIM), 1.0, 0.0).astype(
        jnp.float32
    )


def _attention_aggregate(item_embeddings, g0, g1, agg_W, agg_b):
    batch = item_embeddings.shape[0]
    grid = (batch // BB,)
    return pl.pallas_call(
        _attn_body,
        grid=grid,
        in_specs=[
            pl.BlockSpec((BB, EMB_DIM), lambda i: (i, 0)),
            pl.BlockSpec(
                (1, BB, N_MEMORY // 2, 2 * EMB_DIM), lambda i: (0, i, 0, 0)
            ),
            pl.BlockSpec(
                (1, BB, N_MEMORY // 2, 2 * EMB_DIM), lambda i: (1, i, 0, 0)
            ),
            pl.BlockSpec((N_HOP * EMB_DIM, EMB_DIM), lambda i: (0, 0)),
            pl.BlockSpec((1, EMB_DIM), lambda i: (0, 0)),
            pl.BlockSpec((2 * EMB_DIM, 2 * EMB_DIM), lambda i: (0, 0)),
        ],
        out_specs=pl.BlockSpec((BB, EMB_DIM), lambda i: (i, 0)),
        out_shape=jax.ShapeDtypeStruct((batch, EMB_DIM), jnp.float32),
    )(item_embeddings, g0, g1, agg_W, agg_b, _sel_matrix())


def kernel(item_embeddings, neighbors, protein_embedding, agg_W, agg_b):
    idx_flat = neighbors.astype(jnp.int32).reshape(-1)
    n_prot = protein_embedding.shape[0]
    zeros = jnp.zeros_like(protein_embedding)
    table_eo = jnp.concatenate(
        [
            jnp.concatenate([protein_embedding, zeros], axis=1),
            jnp.concatenate([zeros, protein_embedding], axis=1),
        ],
        axis=0,
    )
    gathered = _sc_gather(
        table_eo, idx_flat[0::2], idx_flat[1::2] + n_prot
    )
    g4 = gathered.reshape(N_HOP, BATCH, N_MEMORY // 2, 2 * EMB_DIM)
    g0 = g4
    g1 = g4
    return _attention_aggregate(
        item_embeddings, g0, g1, agg_W, agg_b.reshape(1, EMB_DIM)
    )


# speedup vs baseline: 5.4690x; 1.0055x over previous
import functools

import jax
import jax.numpy as jnp
from jax import lax
from jax.experimental import pallas as pl
from jax.experimental.pallas import tpu as pltpu
from jax.experimental.pallas import tpu_sc as plsc

N_HOP = 2
N_MEMORY = 32
EMB_DIM = 64
BATCH = 16384
ROWS = N_HOP * BATCH * N_MEMORY
NW = 32
PER_W = ROWS // NW
CHUNK = 512
SUB = 128
N_CHUNK = PER_W // CHUNK

BB = 512


PAIRS_W = PER_W // 2
PCHUNK = 256
NSUB = PCHUNK // SUB
NCH = PAIRS_W // PCHUNK


def _sc_gather_body(table_hbm, idx_ev_hbm, idx_od_hbm, out_hbm, idx_ev_v, idx_od_v, rows_v, gsem, asem, wsem):
    wid = lax.axis_index("s") * 2 + lax.axis_index("c")
    base = wid * PAIRS_W
    pltpu.sync_copy(idx_ev_hbm.at[pl.ds(base, PAIRS_W)], idx_ev_v)
    pltpu.sync_copy(idx_od_hbm.at[pl.ds(base, PAIRS_W)], idx_od_v)

    def e_descs(c, buf):
        return [
            pltpu.make_async_copy(
                table_hbm.at[idx_ev_v.at[pl.ds(c * PCHUNK + s * SUB, SUB)]],
                rows_v.at[buf, pl.ds(s * SUB, SUB)],
                gsem,
            )
            for s in range(NSUB)
        ]

    def o_descs(c, buf):
        return [
            pltpu.make_async_copy(
                table_hbm.at[idx_od_v.at[pl.ds(c * PCHUNK + s * SUB, SUB)]],
                rows_v.at[buf, pl.ds(s * SUB, SUB)],
                asem,
            )
            for s in range(NSUB)
        ]

    def w_desc(c, buf):
        return pltpu.make_async_copy(
            rows_v.at[buf],
            out_hbm.at[pl.ds(base + c * PCHUNK, PCHUNK)],
            wsem,
        )

    def start_e(c, buf):
        for d in e_descs(c, buf):
            d.start()

    def start_o(c, buf):
        for d in o_descs(c, buf):
            d.start(add=True)

    def wait_e(c, buf):
        for d in e_descs(c, buf):
            d.wait()

    def wait_o(c, buf):
        for d in o_descs(c, buf):
            d.wait()

    start_e(0, 0)

    def body(i, carry):
        a = 2 * i
        b = a + 1
        wait_e(a, 0)
        start_o(a, 0)

        @pl.when(i > 0)
        def _():
            w_desc(b - 2, 1).wait()

        start_e(b, 1)
        wait_o(a, 0)
        w_desc(a, 0).start()
        wait_e(b, 1)
        start_o(b, 1)
        w_desc(a, 0).wait()

        @pl.when(i < NCH // 2 - 1)
        def _():
            start_e(a + 2, 0)

        wait_o(b, 1)
        w_desc(b, 1).start()
        return carry

    lax.fori_loop(0, NCH // 2, body, 0)
    w_desc(NCH - 1, 1).wait()


def _sc_gather(table_eo, idx_even, idx_odd):
    mesh = plsc.VectorSubcoreMesh(core_axis_name="c", subcore_axis_name="s")
    kern = functools.partial(
        pl.kernel,
        out_type=jax.ShapeDtypeStruct((ROWS // 2, 2 * EMB_DIM), jnp.float32),
        mesh=mesh,
        scratch_types=[
            pltpu.VMEM((PAIRS_W,), jnp.int32),
            pltpu.VMEM((PAIRS_W,), jnp.int32),
            pltpu.VMEM((2, PCHUNK, 2 * EMB_DIM), jnp.float32),
            pltpu.SemaphoreType.DMA,
            pltpu.SemaphoreType.DMA,
            pltpu.SemaphoreType.DMA,
        ],
    )(_sc_gather_body)
    return kern(table_eo, idx_even, idx_odd)


def _attn_body(item_ref, g_ref, w_ref, b_ref, sel_ref, out_ref):
    x = item_ref[...]
    x2 = jnp.concatenate([x, x], axis=1)
    w = w_ref[...]
    sel = sel_ref[...]

    M2 = N_MEMORY // 2
    D2 = 2 * EMB_DIM

    def hop(g, xin):
        p = (g * xin[:, None, :]).reshape(BB * M2, D2)
        c_wide = lax.dot_general(
            p, sel, (((1,), (0,)), ((), ())),
            precision=lax.Precision.DEFAULT, preferred_element_type=jnp.float32,
        ).reshape(BB, M2, D2)
        e = jnp.exp(c_wide)
        es = jnp.sum(e, axis=1)
        st = es[:, :EMB_DIM] + es[:, EMB_DIM:]
        rinv = 1.0 / st
        r2 = jnp.concatenate([rinv, rinv], axis=1)
        t = jnp.sum(g * e, axis=1) * r2
        return t[:, :EMB_DIM] + t[:, EMB_DIM:]

    i0 = hop(g_ref[0], x2)
    i1 = hop(g_ref[1], jnp.concatenate([i0, i0], axis=1))
    out_ref[...] = (
        lax.dot_general(
            i0, w[:EMB_DIM], (((1,), (0,)), ((), ())),
            precision=lax.Precision.HIGHEST, preferred_element_type=jnp.float32,
        )
        + lax.dot_general(
            i1, w[EMB_DIM:], (((1,), (0,)), ((), ())),
            precision=lax.Precision.HIGHEST, preferred_element_type=jnp.float32,
        )
        + b_ref[...]
    )


def _sel_matrix():
    col = jnp.arange(2 * EMB_DIM)[None, :]
    row = jnp.arange(2 * EMB_DIM)[:, None]
    return jnp.where((col < EMB_DIM) == (row < EMB_DIM), 1.0, 0.0).astype(
        jnp.float32
    )


def _attention_aggregate(item_embeddings, g4, agg_W, agg_b):
    batch = item_embeddings.shape[0]
    grid = (batch // BB,)
    return pl.pallas_call(
        _attn_body,
        grid=grid,
        in_specs=[
            pl.BlockSpec((BB, EMB_DIM), lambda i: (i, 0)),
            pl.BlockSpec(
                (N_HOP, BB, N_MEMORY // 2, 2 * EMB_DIM), lambda i: (0, i, 0, 0)
            ),
            pl.BlockSpec((N_HOP * EMB_DIM, EMB_DIM), lambda i: (0, 0)),
            pl.BlockSpec((1, EMB_DIM), lambda i: (0, 0)),
            pl.BlockSpec((2 * EMB_DIM, 2 * EMB_DIM), lambda i: (0, 0)),
        ],
        out_specs=pl.BlockSpec((BB, EMB_DIM), lambda i: (i, 0)),
        out_shape=jax.ShapeDtypeStruct((batch, EMB_DIM), jnp.float32),
    )(item_embeddings, g4, agg_W, agg_b, _sel_matrix())


def kernel(item_embeddings, neighbors, protein_embedding, agg_W, agg_b):
    idx_flat = neighbors.astype(jnp.int32).reshape(-1)
    n_prot = protein_embedding.shape[0]
    zeros = jnp.zeros_like(protein_embedding)
    table_eo = jnp.concatenate(
        [
            jnp.concatenate([protein_embedding, zeros], axis=1),
            jnp.concatenate([zeros, protein_embedding], axis=1),
        ],
        axis=0,
    )
    gathered = _sc_gather(
        table_eo, idx_flat[0::2], idx_flat[1::2] + n_prot
    )
    g4 = gathered.reshape(N_HOP, BATCH, N_MEMORY // 2, 2 * EMB_DIM)
    return _attention_aggregate(
        item_embeddings, g4, agg_W, agg_b.reshape(1, EMB_DIM)
    )


# speedup vs baseline: 5.9375x; 1.0857x over previous
import functools

import jax
import jax.numpy as jnp
from jax import lax
from jax.experimental import pallas as pl
from jax.experimental.pallas import tpu as pltpu
from jax.experimental.pallas import tpu_sc as plsc

N_HOP = 2
N_MEMORY = 32
EMB_DIM = 64
BATCH = 16384
ROWS = N_HOP * BATCH * N_MEMORY
NW = 32
PER_W = ROWS // NW
CHUNK = 512
SUB = 128
N_CHUNK = PER_W // CHUNK

BB = 512


HROWS = ROWS // N_HOP
PAIRS_W = HROWS // 2 // NW
PCHUNK = 256
NSUB = PCHUNK // SUB
NCH = PAIRS_W // PCHUNK


def _sc_gather_body(table_hbm, idx_ev_hbm, idx_od_hbm, out_hbm, idx_ev_v, idx_od_v, rows_v, gsem, asem, wsem):
    wid = lax.axis_index("s") * 2 + lax.axis_index("c")
    base = wid * PAIRS_W
    pltpu.sync_copy(idx_ev_hbm.at[pl.ds(base, PAIRS_W)], idx_ev_v)
    pltpu.sync_copy(idx_od_hbm.at[pl.ds(base, PAIRS_W)], idx_od_v)

    def e_descs(c, buf):
        return [
            pltpu.make_async_copy(
                table_hbm.at[idx_ev_v.at[pl.ds(c * PCHUNK + s * SUB, SUB)]],
                rows_v.at[buf, pl.ds(s * SUB, SUB)],
                gsem,
            )
            for s in range(NSUB)
        ]

    def o_descs(c, buf):
        return [
            pltpu.make_async_copy(
                table_hbm.at[idx_od_v.at[pl.ds(c * PCHUNK + s * SUB, SUB)]],
                rows_v.at[buf, pl.ds(s * SUB, SUB)],
                asem,
            )
            for s in range(NSUB)
        ]

    def w_desc(c, buf):
        return pltpu.make_async_copy(
            rows_v.at[buf],
            out_hbm.at[pl.ds(base + c * PCHUNK, PCHUNK)],
            wsem,
        )

    def start_e(c, buf):
        for d in e_descs(c, buf):
            d.start()

    def start_o(c, buf):
        for d in o_descs(c, buf):
            d.start(add=True)

    def wait_e(c, buf):
        for d in e_descs(c, buf):
            d.wait()

    def wait_o(c, buf):
        for d in o_descs(c, buf):
            d.wait()

    start_e(0, 0)

    def body(i, carry):
        a = 2 * i
        b = a + 1
        wait_e(a, 0)
        start_o(a, 0)

        @pl.when(i > 0)
        def _():
            w_desc(b - 2, 1).wait()

        start_e(b, 1)
        wait_o(a, 0)
        w_desc(a, 0).start()
        wait_e(b, 1)
        start_o(b, 1)
        w_desc(a, 0).wait()

        @pl.when(i < NCH // 2 - 1)
        def _():
            start_e(a + 2, 0)

        wait_o(b, 1)
        w_desc(b, 1).start()
        return carry

    lax.fori_loop(0, NCH // 2, body, 0)
    w_desc(NCH - 1, 1).wait()


def _sc_gather(table_eo, idx_even, idx_odd):
    mesh = plsc.VectorSubcoreMesh(core_axis_name="c", subcore_axis_name="s")
    kern = functools.partial(
        pl.kernel,
        out_type=jax.ShapeDtypeStruct((HROWS // 2, 2 * EMB_DIM), jnp.float32),
        mesh=mesh,
        scratch_types=[
            pltpu.VMEM((PAIRS_W,), jnp.int32),
            pltpu.VMEM((PAIRS_W,), jnp.int32),
            pltpu.VMEM((2, PCHUNK, 2 * EMB_DIM), jnp.float32),
            pltpu.SemaphoreType.DMA,
            pltpu.SemaphoreType.DMA,
            pltpu.SemaphoreType.DMA,
        ],
    )(_sc_gather_body)
    return kern(table_eo, idx_even, idx_odd)


def _hop_math(g, xin, sel):
    M2 = N_MEMORY // 2
    D2 = 2 * EMB_DIM
    p = (g * xin[:, None, :]).reshape(g.shape[0] * M2, D2)
    c_wide = lax.dot_general(
        p, sel, (((1,), (0,)), ((), ())),
        precision=lax.Precision.DEFAULT, preferred_element_type=jnp.float32,
    ).reshape(g.shape[0], M2, D2)
    e = jnp.exp(c_wide)
    es = jnp.sum(e, axis=1)
    st = es[:, :EMB_DIM] + es[:, EMB_DIM:]
    rinv = 1.0 / st
    r2 = jnp.concatenate([rinv, rinv], axis=1)
    t = jnp.sum(g * e, axis=1) * r2
    return t[:, :EMB_DIM] + t[:, EMB_DIM:]


def _hop_body(item_ref, g_ref, sel_ref, out_ref):
    x = item_ref[...]
    x2 = jnp.concatenate([x, x], axis=1)
    out_ref[...] = _hop_math(g_ref[...], x2, sel_ref[...])


def _final_body(i0_ref, g_ref, w_ref, b_ref, sel_ref, out_ref):
    i0 = i0_ref[...]
    i1 = _hop_math(
        g_ref[...], jnp.concatenate([i0, i0], axis=1), sel_ref[...]
    )
    w = w_ref[...]
    out_ref[...] = (
        lax.dot_general(
            i0, w[:EMB_DIM], (((1,), (0,)), ((), ())),
            precision=lax.Precision.HIGHEST, preferred_element_type=jnp.float32,
        )
        + lax.dot_general(
            i1, w[EMB_DIM:], (((1,), (0,)), ((), ())),
            precision=lax.Precision.HIGHEST, preferred_element_type=jnp.float32,
        )
        + b_ref[...]
    )


def _sel_matrix():
    col = jnp.arange(2 * EMB_DIM)[None, :]
    row = jnp.arange(2 * EMB_DIM)[:, None]
    return jnp.where((col < EMB_DIM) == (row < EMB_DIM), 1.0, 0.0).astype(
        jnp.float32
    )


_G_SPEC = lambda: pl.BlockSpec(
    (BB, N_MEMORY // 2, 2 * EMB_DIM), lambda i: (i, 0, 0)
)
_V_SPEC = lambda: pl.BlockSpec((BB, EMB_DIM), lambda i: (i, 0))
_SEL_SPEC = lambda: pl.BlockSpec((2 * EMB_DIM, 2 * EMB_DIM), lambda i: (0, 0))


def _attention_hop(item_embeddings, g):
    batch = item_embeddings.shape[0]
    return pl.pallas_call(
        _hop_body,
        grid=(batch // BB,),
        in_specs=[_V_SPEC(), _G_SPEC(), _SEL_SPEC()],
        out_specs=_V_SPEC(),
        out_shape=jax.ShapeDtypeStruct((batch, EMB_DIM), jnp.float32),
    )(item_embeddings, g, _sel_matrix())


def _attention_final(i0, g, agg_W, agg_b):
    batch = i0.shape[0]
    return pl.pallas_call(
        _final_body,
        grid=(batch // BB,),
        in_specs=[
            _V_SPEC(),
            _G_SPEC(),
            pl.BlockSpec((N_HOP * EMB_DIM, EMB_DIM), lambda i: (0, 0)),
            pl.BlockSpec((1, EMB_DIM), lambda i: (0, 0)),
            _SEL_SPEC(),
        ],
        out_specs=_V_SPEC(),
        out_shape=jax.ShapeDtypeStruct((batch, EMB_DIM), jnp.float32),
    )(i0, g, agg_W, agg_b, _sel_matrix())


def kernel(item_embeddings, neighbors, protein_embedding, agg_W, agg_b):
    idx = neighbors.astype(jnp.int32).reshape(N_HOP, HROWS)
    n_prot = protein_embedding.shape[0]
    zeros = jnp.zeros_like(protein_embedding)
    table_eo = jnp.concatenate(
        [
            jnp.concatenate([protein_embedding, zeros], axis=1),
            jnp.concatenate([zeros, protein_embedding], axis=1),
        ],
        axis=0,
    )
    g_hops = [
        _sc_gather(table_eo, idx[h, 0::2], idx[h, 1::2] + n_prot).reshape(
            BATCH, N_MEMORY // 2, 2 * EMB_DIM
        )
        for h in range(N_HOP)
    ]
    i0 = _attention_hop(item_embeddings, g_hops[0])
    return _attention_final(
        i0, g_hops[1], agg_W, agg_b.reshape(1, EMB_DIM)
    )
